# baseline (speedup 1.0000x reference)
import functools

import jax
import jax.numpy as jnp
import numpy as np
from jax import lax
from jax.experimental import pallas as pl
from jax.experimental.pallas import tpu as pltpu
from jax.experimental.pallas import tpu_sc as plsc

_B = 32
_T = 160000
_CS = 16000
_CPR = _T // _CS
_NR = 2 * _B
_NC_ROWS = _NR * _CPR
_NW = 32
_KPW = _NC_ROWS // _NW
_G = 2
_NG = _KPW // _G

def _chunk_indices():
    perm = jnp.argsort(jax.random.uniform(jax.random.key(42), (_B,)), axis=0)
    rows = jnp.concatenate([perm, jnp.arange(_B, _NR)])
    cidx = rows[:, None] * _CPR + jnp.arange(_CPR)[None, :]
    return cidx.reshape(_NW, _NG, _G).astype(jnp.int32)


_mesh = plsc.VectorSubcoreMesh(core_axis_name="c", subcore_axis_name="s")


@functools.partial(
    pl.kernel,
    out_type=jax.ShapeDtypeStruct((_NC_ROWS, _CS), jnp.float32),
    mesh=_mesh,
    scratch_types=[
        pltpu.VMEM((_NG, _G), jnp.int32),
        pltpu.VMEM((_G, _CS), jnp.float32),
        pltpu.VMEM((_G, _CS), jnp.float32),
        pltpu.SemaphoreType.DMA,
        pltpu.SemaphoreType.DMA,
    ],
)
def _remix_copy(src, cidx, out, idx_v, buf0, buf1, sem0, sem1):
    wid = lax.axis_index("s") * 2 + lax.axis_index("c")
    base = wid * _KPW
    pltpu.sync_copy(cidx.at[wid], idx_v)
    bufs = (buf0, buf1)
    sems = (sem0, sem1)
    gathers = [None] * _NG
    gathers[0] = pltpu.async_copy(src.at[idx_v.at[0]], bufs[0], sems[0])
    for g in range(_NG):
        if g + 1 < _NG:
            nb = (g + 1) % 2
            gathers[g + 1] = pltpu.async_copy(
                src.at[idx_v.at[g + 1]], bufs[nb], sems[nb]
            )
        gathers[g].wait()
        pltpu.sync_copy(bufs[g % 2], out.at[pl.ds(base + g * _G, _G)])


def kernel(sources, target):
    src = sources.reshape(_NC_ROWS, _CS)
    out = _remix_copy(src, _chunk_indices())
    return out.reshape(2, _B, 1, _T), target
